# Initial kernel scaffold; baseline (speedup 1.0000x reference)
import jax
import jax.numpy as jnp
from jax.experimental import pallas as pl


def kernel(node_features, node_types, edge_index, edge_weights, query_pairs, enc_W, enc_b, ln_g, ln_b, W1, a1, W2, a2, W3, a3, res_W, res_b, p1_W, p1_b, p2_W, p2_b, p3_W, p3_b):
    raise NotImplementedError("write your pallas kernel here")


# speedup vs baseline: 1.0894x; 1.0894x over previous
import jax
import jax.numpy as jnp
from jax.experimental import pallas as pl

N = 10000
E = 320000
D = 128
HID = 64
HEADS = 4
Q = 4096


def _mlp_body(pair_ref, p1w_ref, p1b_ref, p2w_ref, p2b_ref, out_ref):
    z = jnp.maximum(jnp.dot(pair_ref[...], p1w_ref[...],
                            preferred_element_type=jnp.float32) + p1b_ref[...], 0.0)
    out_ref[...] = jnp.maximum(jnp.dot(z, p2w_ref[...],
                               preferred_element_type=jnp.float32) + p2b_ref[...], 0.0)


def kernel(node_features, node_types, edge_index, edge_weights, query_pairs,
           enc_W, enc_b, ln_g, ln_b, W1, a1, W2, a2, W3, a3,
           res_W, res_b, p1_W, p1_b, p2_W, p2_b, p3_W, p3_b):
    n = node_features.shape[0]
    src, dst = edge_index[0], edge_index[1]

    h = jnp.zeros((n, D), dtype=jnp.float32)
    for t in range(3):
        y = node_features @ enc_W[t] + enc_b[t]
        mu = jnp.mean(y, axis=-1, keepdims=True)
        var = jnp.var(y, axis=-1, keepdims=True)
        y = (y - mu) / jnp.sqrt(var + 1e-5) * ln_g[t] + ln_b[t]
        y = jnp.maximum(y, 0.0)
        h = jnp.where((node_types == t)[:, None], y, h)

    h_residual = h @ res_W + res_b
    ew1 = 1.0 + jnp.clip(edge_weights, 0.0, None)

    for (W, a, concat) in ((W1, a1, True), (W2, a2, True), (W3, a3, False)):
        outs = []
        for k in range(HEADS):
            hp = h @ W[k]
            s_src = hp @ a[k, :HID]
            s_dst = hp @ a[k, HID:]
            e = s_src[src] + s_dst[dst]
            e = jnp.where(e > 0, e, 0.2 * e)
            exp_e = jnp.exp(e) * ew1
            sum_exp = jnp.zeros((n,), jnp.float32).at[dst].add(exp_e)
            num = jnp.zeros((n, HID), jnp.float32).at[dst].add(exp_e[:, None] * hp[src])
            h_new = num / (sum_exp[:, None] + 1e-8)
            outs.append(jnp.where(h_new > 0, h_new, jnp.expm1(h_new)))
        if concat:
            h = jnp.concatenate(outs, axis=-1)
        else:
            h = jnp.mean(jnp.stack(outs, 0), 0) + h_residual

    h_ad = h[query_pairs[:, 0]]
    h_cat = h[query_pairs[:, 1]]
    pair = jnp.concatenate([h_ad, h_cat], axis=-1)
    z = pl.pallas_call(
        _mlp_body,
        out_shape=jax.ShapeDtypeStruct((Q, 64), jnp.float32),
    )(pair, p1_W, p1_b[None, :], p2_W, p2_b[None, :])
    logits = (z @ p3_W + p3_b)[:, 0]
    return jax.nn.sigmoid(logits)


# speedup vs baseline: 21.0117x; 19.2881x over previous
import functools

import jax
import jax.numpy as jnp
from jax import lax
from jax.experimental import pallas as pl
from jax.experimental.pallas import tpu as pltpu
from jax.experimental.pallas import tpu_sc as plsc

N = 10000
E = 320000
D = 128
HID = 64
HEADS = 4
Q = 4096

NC = 2
NS = 16
L = 16

EP = E // NS
CHUNK = 4000
B = 80
BPC = CHUNK // B
PAIRS = BPC // 2
NP = 10240
NPT = NP // NS
CPY = 64
_mesh = plsc.VectorSubcoreMesh(core_axis_name="c", subcore_axis_name="s")
_sc_params = pltpu.CompilerParams(needs_layout_passes=False,
                                  use_tc_tiling_on_sc=False)


def _edge_body(hp_hbm, sp_hbm, src_hbm, dst_hbm, ew_hbm, out_hbm,
               sv, src_c, dst_c, ew_c,
               idxs0, idxd0, expv0, gbuf0, obuf0, dbuf0, didx0,
               idxs1, idxd1, expv1, gbuf1, obuf1, dbuf1, didx1,
               zbuf, dzv, accS, accD,
               gsem0, gsem1, ssem0, ssem1, dsem0, dsem1):
    c = lax.axis_index("c")
    s = lax.axis_index("s")

    slots = ((idxs0, idxd0, expv0, gbuf0, obuf0, dbuf0, didx0, gsem0,
              ssem0, dsem0),
             (idxs1, idxd1, expv1, gbuf1, obuf1, dbuf1, didx1, gsem1,
              ssem1, dsem1))

    def _pass(j, _):
        pltpu.sync_copy(sp_hbm.at[c, pl.ds(j * 2 * N, 2 * N)], sv)
        def _z(i, _):
            for q in range(HID // L):
                zbuf[i, pl.ds(q * L, L)] = jnp.zeros((L,), jnp.float32)
            return 0
        lax.fori_loop(0, CPY, _z, 0)
        for q in range(NPT // L):
            dzv[pl.ds(q * L, L)] = jnp.zeros((L,), jnp.float32)
        for k in range(NPT // CPY):
            pltpu.sync_copy(zbuf, accS.at[pl.ds(s * NPT + k * CPY, CPY)])
        pltpu.sync_copy(dzv, accD.at[pl.ds(s * NPT, NPT)])
        plsc.subcore_barrier()

        hb = (2 * c + j) * N

        def scalar_phase(slot, b):
            idxs_, idxd_, expv_ = slots[slot][0], slots[slot][1], slots[slot][2]
            for g in range(B // L):
                i0 = b * B + g * L
                srcv = src_c[pl.ds(i0, L)]
                dstv = dst_c[pl.ds(i0, L)]
                ewv = ew_c[pl.ds(i0, L)]
                idxs_[pl.ds(g * L, L)] = srcv + hb
                idxd_[pl.ds(g * L, L)] = dstv
                ssrc = plsc.load_gather(sv, [srcv])
                sdst = plsc.load_gather(sv, [dstv + N])
                e = ssrc + sdst
                e = jnp.where(e > 0, e, 0.2 * e)
                expv_[pl.ds(g * L, L)] = (
                    jnp.exp(e) * (1.0 + jnp.maximum(ewv, 0.0)))

        def gather_start(slot):
            pltpu.async_copy(hp_hbm.at[slots[slot][0]], slots[slot][3],
                             slots[slot][7])

        def gather_wait(slot):
            pltpu.make_async_copy(hp_hbm.at[slots[slot][0]], slots[slot][3],
                                  slots[slot][7]).wait()

        def scale(slot):
            (_, idxd_, expv_, gbuf_, obuf_, dbuf_, didx_) = slots[slot][:7]

            def _row16(g, _):
                ev = expv_[pl.ds(g * L, L)]
                dbuf_[pl.ds(g * L, L)] = ev
                didx_[pl.ds(g * L, L)] = idxd_[pl.ds(g * L, L)]
                for t in range(L):
                    i = g * L + t
                    e0 = ev[t]
                    for q in range(HID // L):
                        obuf_[i, pl.ds(q * L, L)] = (
                            gbuf_[i, pl.ds(q * L, L)] * e0)
                return 0
            lax.fori_loop(0, B // L, _row16, 0)

        def scatter_start(slot):
            pltpu.async_copy(slots[slot][4], accS.at[slots[slot][1]],
                             slots[slot][8], add=True)
            pltpu.async_copy(slots[slot][5], accD.at[slots[slot][6]],
                             slots[slot][9], add=True)

        def scatter_wait(slot):
            pltpu.make_async_copy(slots[slot][4], accS.at[slots[slot][1]],
                                  slots[slot][8]).wait()
            pltpu.make_async_copy(slots[slot][5], accD.at[slots[slot][6]],
                                  slots[slot][9]).wait()

        def _chunk(ch, _):
            off = s * EP + ch * CHUNK
            pltpu.sync_copy(src_hbm.at[pl.ds(off, CHUNK)], src_c)
            pltpu.sync_copy(dst_hbm.at[pl.ds(off, CHUNK)], dst_c)
            pltpu.sync_copy(ew_hbm.at[pl.ds(off, CHUNK)], ew_c)

            scalar_phase(0, 0)
            gather_start(0)

            def _pair(t, _):
                b0 = 2 * t
                scalar_phase(1, b0 + 1)
                gather_start(1)
                gather_wait(0)

                @pl.when(t > 0)
                def _():
                    scatter_wait(0)
                scale(0)
                scatter_start(0)

                @pl.when(t < PAIRS - 1)
                def _():
                    scalar_phase(0, b0 + 2)
                    gather_start(0)

                gather_wait(1)

                @pl.when(t > 0)
                def _():
                    scatter_wait(1)
                scale(1)
                scatter_start(1)
                return 0
            lax.fori_loop(0, PAIRS, _pair, 0)
            scatter_wait(0)
            scatter_wait(1)
            return 0
        lax.fori_loop(0, EP // CHUNK, _chunk, 0)

        plsc.subcore_barrier()
        base = (2 * c + j) * NP
        pltpu.sync_copy(accD.at[pl.ds(s * NPT, NPT)], dzv)
        for k in range(NPT // CPY):
            row0 = s * NPT + k * CPY
            pltpu.sync_copy(accS.at[pl.ds(row0, CPY)], zbuf)

            def _div16(g, _):
                dv = dzv[pl.ds(k * CPY + g * L, L)]
                rv = 1.0 / (dv + 1e-8)
                for t in range(L):
                    i = g * L + t
                    r0 = rv[t]
                    for q in range(HID // L):
                        zbuf[i, pl.ds(q * L, L)] = (
                            zbuf[i, pl.ds(q * L, L)] * r0)
                return 0
            lax.fori_loop(0, CPY // L, _div16, 0)
            pltpu.sync_copy(zbuf, out_hbm.at[pl.ds(base + row0, CPY)])
        return 0
    lax.fori_loop(0, 2, _pass, 0)


@functools.partial(
    pl.kernel,
    out_type=jax.ShapeDtypeStruct((4 * NP, HID), jnp.float32),
    mesh=_mesh,
    scratch_types=[
        pltpu.VMEM((2 * N,), jnp.float32),
        pltpu.VMEM((CHUNK,), jnp.int32),
        pltpu.VMEM((CHUNK,), jnp.int32),
        pltpu.VMEM((CHUNK,), jnp.float32),
        pltpu.VMEM((B,), jnp.int32),
        pltpu.VMEM((B,), jnp.int32),
        pltpu.VMEM((B,), jnp.float32),
        pltpu.VMEM((B, HID), jnp.float32),
        pltpu.VMEM((B, HID), jnp.float32),
        pltpu.VMEM((B,), jnp.float32),
        pltpu.VMEM((B,), jnp.int32),
        pltpu.VMEM((B,), jnp.int32),
        pltpu.VMEM((B,), jnp.int32),
        pltpu.VMEM((B,), jnp.float32),
        pltpu.VMEM((B, HID), jnp.float32),
        pltpu.VMEM((B, HID), jnp.float32),
        pltpu.VMEM((B,), jnp.float32),
        pltpu.VMEM((B,), jnp.int32),
        pltpu.VMEM((CPY, HID), jnp.float32),
        pltpu.VMEM((NPT,), jnp.float32),
        pltpu.VMEM_SHARED((NP, HID), jnp.float32),
        pltpu.VMEM_SHARED((NP,), jnp.float32),
        pltpu.SemaphoreType.DMA,
        pltpu.SemaphoreType.DMA,
        pltpu.SemaphoreType.DMA,
        pltpu.SemaphoreType.DMA,
        pltpu.SemaphoreType.DMA,
        pltpu.SemaphoreType.DMA,
    ],
    compiler_params=_sc_params,
)
def _edge_kernel(*refs):
    _edge_body(*refs)


QPT = Q // (NC * NS)


def _qgather_body(h_hbm, qa_hbm, qb_hbm, oa_hbm, ob_hbm, idxv, buf, sem):
    c = lax.axis_index("c")
    s = lax.axis_index("s")
    base = (s * NC + c) * QPT
    pltpu.sync_copy(qa_hbm.at[pl.ds(base, QPT)], idxv)
    pltpu.async_copy(h_hbm.at[idxv], buf, sem).wait()
    pltpu.sync_copy(buf, oa_hbm.at[pl.ds(base, QPT)])
    pltpu.sync_copy(qb_hbm.at[pl.ds(base, QPT)], idxv)
    pltpu.async_copy(h_hbm.at[idxv], buf, sem).wait()
    pltpu.sync_copy(buf, ob_hbm.at[pl.ds(base, QPT)])


@functools.partial(
    pl.kernel,
    out_type=(jax.ShapeDtypeStruct((Q, HID), jnp.float32),
              jax.ShapeDtypeStruct((Q, HID), jnp.float32)),
    mesh=_mesh,
    scratch_types=[
        pltpu.VMEM((QPT,), jnp.int32),
        pltpu.VMEM((QPT, HID), jnp.float32),
        pltpu.SemaphoreType.DMA,
    ],
    compiler_params=_sc_params,
)
def _qgather_kernel(*refs):
    _qgather_body(*refs)


def _mlp_body(ha_ref, hb_ref, p1wa_ref, p1wb_ref, p1b_ref, p2w_ref, p2b_ref,
              p3w_ref, p3b_ref, out_ref):
    z = jnp.dot(ha_ref[...], p1wa_ref[...], preferred_element_type=jnp.float32)
    z = z + jnp.dot(hb_ref[...], p1wb_ref[...],
                    preferred_element_type=jnp.float32)
    z = jnp.maximum(z + p1b_ref[...], 0.0)
    z = jnp.maximum(jnp.dot(z, p2w_ref[...],
                            preferred_element_type=jnp.float32) + p2b_ref[...],
                    0.0)
    logits = jnp.sum(z * p3w_ref[...], axis=1, keepdims=True)
    out_ref[...] = jax.nn.sigmoid(logits + p3b_ref[0:1, 0:1])


NB = 1000
GRID = N // NB


def _pad8(m):
    return jnp.zeros((8, m.shape[-1]), jnp.float32).at[:m.shape[0]].set(m)


def _enc_body(x_ref, nt_ref, encW_ref, encb_ref, lng_ref, lnb_ref,
              resW_ref, resb_ref, h_ref, hres_ref):
    x = x_ref[...]
    nt = nt_ref[...]
    acc = jnp.zeros((NB, D), jnp.float32)
    for t in range(3):
        y = jnp.dot(x, encW_ref[t], preferred_element_type=jnp.float32)
        y = y + encb_ref[t:t + 1, :]
        mu = jnp.mean(y, axis=-1, keepdims=True)
        dev = y - mu
        var = jnp.mean(dev * dev, axis=-1, keepdims=True)
        yn = dev * jax.lax.rsqrt(var + 1e-5)
        yn = yn * lng_ref[t:t + 1, :] + lnb_ref[t:t + 1, :]
        yn = jnp.maximum(yn, 0.0)
        acc = jnp.where(nt == t, yn, acc)
    h_ref[...] = acc
    hres_ref[...] = (jnp.dot(acc, resW_ref[...],
                             preferred_element_type=jnp.float32)
                     + resb_ref[0:1, :])


def _encode(x, nt, enc_W, enc_b, ln_g, ln_b, res_W, res_b):
    return pl.pallas_call(
        _enc_body,
        grid=(GRID,),
        in_specs=[
            pl.BlockSpec((NB, D), lambda i: (i, 0)),
            pl.BlockSpec((NB, 1), lambda i: (i, 0)),
            pl.BlockSpec((3, D, D), lambda i: (0, 0, 0)),
            pl.BlockSpec((8, D), lambda i: (0, 0)),
            pl.BlockSpec((8, D), lambda i: (0, 0)),
            pl.BlockSpec((8, D), lambda i: (0, 0)),
            pl.BlockSpec((D, HID), lambda i: (0, 0)),
            pl.BlockSpec((8, HID), lambda i: (0, 0)),
        ],
        out_specs=[pl.BlockSpec((NB, D), lambda i: (i, 0)),
                   pl.BlockSpec((NB, HID), lambda i: (i, 0))],
        out_shape=[jax.ShapeDtypeStruct((N, D), jnp.float32),
                   jax.ShapeDtypeStruct((N, HID), jnp.float32)],
    )(x, nt, enc_W, _pad8(enc_b), _pad8(ln_g), _pad8(ln_b), res_W,
      _pad8(res_b[None, :]))


def _pre_body(h_ref, W_ref, a_ref, hp_ref, s_ref):
    h = h_ref[...]
    for hd in range(HEADS):
        y = jnp.dot(h, W_ref[hd], preferred_element_type=jnp.float32)
        hp_ref[hd] = y
        asrc = a_ref[hd:hd + 1, :HID]
        adst = a_ref[hd:hd + 1, HID:]
        s_ref[0, 2 * hd:2 * hd + 1, :] = lax.dot_general(
            asrc, y, (((1,), (1,)), ((), ())))
        s_ref[0, 2 * hd + 1:2 * hd + 2, :] = lax.dot_general(
            adst, y, (((1,), (1,)), ((), ())))


def _layer_pre(h, W, a):
    din = h.shape[1]
    hp4, s8 = pl.pallas_call(
        _pre_body,
        grid=(GRID,),
        in_specs=[
            pl.BlockSpec((NB, din), lambda i: (i, 0)),
            pl.BlockSpec((HEADS, din, HID), lambda i: (0, 0, 0)),
            pl.BlockSpec((8, 2 * HID), lambda i: (0, 0)),
        ],
        out_specs=[pl.BlockSpec((HEADS, NB, HID), lambda i: (0, i, 0)),
                   pl.BlockSpec((1, 8, NB), lambda i: (i, 0, 0))],
        out_shape=[jax.ShapeDtypeStruct((HEADS, N, HID), jnp.float32),
                   jax.ShapeDtypeStruct((GRID, 8, NB), jnp.float32)],
    )(h, W, _pad8(a))
    s8 = s8.transpose(1, 0, 2).reshape(8, N)
    return hp4.reshape(HEADS * N, HID), s8.reshape(2, 4 * N)


def _post_cat_body(acc_ref, h_ref):
    for hd in range(HEADS):
        hn = acc_ref[hd]
        hn = jnp.where(hn > 0, hn, jnp.exp(hn) - 1.0)
        h_ref[:, hd * HID:(hd + 1) * HID] = hn


def _post_mean_body(acc_ref, hres_ref, h_ref):
    total = jnp.zeros((NB, HID), jnp.float32)
    for hd in range(HEADS):
        hn = acc_ref[hd]
        hn = jnp.where(hn > 0, hn, jnp.exp(hn) - 1.0)
        total = total + hn
    h_ref[...] = total * (1.0 / HEADS) + hres_ref[...]


def _layer_post(acc, hres):
    if hres is None:
        return pl.pallas_call(
            _post_cat_body,
            grid=(GRID,),
            in_specs=[pl.BlockSpec((HEADS, NB, HID), lambda i: (0, i, 0))],
            out_specs=pl.BlockSpec((NB, HEADS * HID), lambda i: (i, 0)),
            out_shape=jax.ShapeDtypeStruct((N, HEADS * HID), jnp.float32),
        )(acc)
    return pl.pallas_call(
        _post_mean_body,
        grid=(GRID,),
        in_specs=[pl.BlockSpec((HEADS, NB, HID), lambda i: (0, i, 0)),
                  pl.BlockSpec((NB, HID), lambda i: (i, 0))],
        out_specs=pl.BlockSpec((NB, HID), lambda i: (i, 0)),
        out_shape=jax.ShapeDtypeStruct((N, HID), jnp.float32),
    )(acc, hres)


def kernel(node_features, node_types, edge_index, edge_weights, query_pairs,
           enc_W, enc_b, ln_g, ln_b, W1, a1, W2, a2, W3, a3,
           res_W, res_b, p1_W, p1_b, p2_W, p2_b, p3_W, p3_b):
    src = edge_index[0].astype(jnp.int32)
    dst = edge_index[1].astype(jnp.int32)
    ew = edge_weights.astype(jnp.float32)
    nt = node_types.astype(jnp.int32).reshape(N, 1)

    h, h_residual = _encode(node_features, nt, enc_W, enc_b, ln_g, ln_b,
                            res_W, res_b)

    for (W, a, concat) in ((W1, a1, True), (W2, a2, True), (W3, a3, False)):
        hp4n, s_pack = _layer_pre(h, W, a)
        acc = _edge_kernel(hp4n, s_pack, src, dst, ew)
        acc = acc.reshape(HEADS, NP, HID)
        h = _layer_post(acc, None if concat else h_residual)

    qa = query_pairs[:, 0].astype(jnp.int32)
    qb = query_pairs[:, 1].astype(jnp.int32)
    h_ad, h_cat = _qgather_kernel(h, qa, qb)

    p3b_p = jnp.zeros((8, 128), jnp.float32).at[0, 0].set(p3_b[0])
    out = pl.pallas_call(
        _mlp_body,
        out_shape=jax.ShapeDtypeStruct((Q, 1), jnp.float32),
    )(h_ad, h_cat, p1_W[:HID], p1_W[HID:], p1_b[None, :], p2_W, p2_b[None, :],
      p3_W.reshape(1, HID), p3b_p)
    return out[:, 0]


# speedup vs baseline: 50.3625x; 2.3969x over previous
import functools

import jax
import jax.numpy as jnp
from jax import lax
from jax.experimental import pallas as pl
from jax.experimental.pallas import tpu as pltpu
from jax.experimental.pallas import tpu_sc as plsc

N = 10000
E = 320000
D = 128
HID = 64
HEADS = 4
Q = 4096

NC = 2
NS = 16
L = 16

EP = E // NS
CHUNK = 4000
B = 80
BPC = CHUNK // B
PAIRS = BPC // 2
NP = 10240
NPT = NP // NS
CPY = 64
DW = 64
DSH = 6
NR = NP // DW
NRT = NR // NS
_mesh = plsc.VectorSubcoreMesh(core_axis_name="c", subcore_axis_name="s")
_sc_params = pltpu.CompilerParams(needs_layout_passes=False,
                                  use_tc_tiling_on_sc=False)


def _edge_body(hp_hbm, sp_hbm, src_hbm, dst_hbm, ew_hbm, out_hbm,
               sv, src_c, dst_c, ew_c,
               idxs0, idxd0, expv0, gbuf0, obuf0,
               idxs1, idxd1, expv1, gbuf1, obuf1,
               zbuf, dzv, denT, idxr, accS, accD,
               gsem0, gsem1, ssem0, ssem1, dsem):
    c = lax.axis_index("c")
    s = lax.axis_index("s")

    slots = ((idxs0, idxd0, expv0, gbuf0, obuf0, gsem0, ssem0),
             (idxs1, idxd1, expv1, gbuf1, obuf1, gsem1, ssem1))

    lane = lax.broadcasted_iota(jnp.int32, (L,), 0)
    for q in range(NR // L):
        idxr[pl.ds(q * L, L)] = lane + q * L

    def _pass(j, _):
        pltpu.sync_copy(sp_hbm.at[c, pl.ds(j * 2 * N, 2 * N)], sv)
        def _z(i, _):
            for q in range(HID // L):
                zbuf[i, pl.ds(q * L, L)] = jnp.zeros((L,), jnp.float32)
            return 0
        lax.fori_loop(0, CPY, _z, 0)
        def _zdz(i, _):
            for q in range(DW // L):
                dzv[i, pl.ds(q * L, L)] = jnp.zeros((L,), jnp.float32)
            return 0
        lax.fori_loop(0, NRT, _zdz, 0)
        for k in range(NPT // CPY):
            pltpu.sync_copy(zbuf, accS.at[pl.ds(s * NPT + k * CPY, CPY)])
        pltpu.sync_copy(dzv, accD.at[pl.ds(s * NRT, NRT)])

        def _zd(i, _):
            for q in range(DW // L):
                denT[i, pl.ds(q * L, L)] = jnp.zeros((L,), jnp.float32)
            return 0
        lax.fori_loop(0, NR, _zd, 0)
        plsc.subcore_barrier()

        hb = (2 * c + j) * N

        def scalar_phase(slot, b):
            idxs_, idxd_, expv_ = slots[slot][0], slots[slot][1], slots[slot][2]
            for g in range(B // L):
                i0 = b * B + g * L
                srcv = src_c[pl.ds(i0, L)]
                dstv = dst_c[pl.ds(i0, L)]
                ewv = ew_c[pl.ds(i0, L)]
                idxs_[pl.ds(g * L, L)] = srcv + hb
                idxd_[pl.ds(g * L, L)] = dstv
                ssrc = plsc.load_gather(sv, [srcv])
                sdst = plsc.load_gather(sv, [dstv + N])
                e = ssrc + sdst
                e = jnp.where(e > 0, e, 0.2 * e)
                ex = jnp.exp(e) * (1.0 + jnp.maximum(ewv, 0.0))
                expv_[pl.ds(g * L, L)] = ex
                plsc.addupdate_scatter(
                    denT,
                    [lax.shift_right_logical(dstv, DSH),
                     jnp.bitwise_and(dstv, DW - 1)], ex)

        def gather_start(slot):
            pltpu.async_copy(hp_hbm.at[slots[slot][0]], slots[slot][3],
                             slots[slot][5])

        def gather_wait(slot):
            pltpu.make_async_copy(hp_hbm.at[slots[slot][0]], slots[slot][3],
                                  slots[slot][5]).wait()

        def scale(slot):
            expv_, gbuf_, obuf_ = slots[slot][2], slots[slot][3], slots[slot][4]

            def _row16(g, _):
                ev = expv_[pl.ds(g * L, L)]
                for t in range(L):
                    i = g * L + t
                    e0 = ev[t]
                    for q in range(HID // L):
                        obuf_[i, pl.ds(q * L, L)] = (
                            gbuf_[i, pl.ds(q * L, L)] * e0)
                return 0
            lax.fori_loop(0, B // L, _row16, 0)

        def scatter_start(slot):
            pltpu.async_copy(slots[slot][4], accS.at[slots[slot][1]],
                             slots[slot][6], add=True)

        def scatter_wait(slot):
            pltpu.make_async_copy(slots[slot][4], accS.at[slots[slot][1]],
                                  slots[slot][6]).wait()

        def _chunk(ch, _):
            off = s * EP + ch * CHUNK
            pltpu.sync_copy(src_hbm.at[pl.ds(off, CHUNK)], src_c)
            pltpu.sync_copy(dst_hbm.at[pl.ds(off, CHUNK)], dst_c)
            pltpu.sync_copy(ew_hbm.at[pl.ds(off, CHUNK)], ew_c)

            scalar_phase(0, 0)
            gather_start(0)

            def _pair(t, _):
                b0 = 2 * t
                scalar_phase(1, b0 + 1)
                gather_start(1)
                gather_wait(0)

                @pl.when(t > 0)
                def _():
                    scatter_wait(0)
                scale(0)
                scatter_start(0)

                @pl.when(t < PAIRS - 1)
                def _():
                    scalar_phase(0, b0 + 2)
                    gather_start(0)

                gather_wait(1)

                @pl.when(t > 0)
                def _():
                    scatter_wait(1)
                scale(1)
                scatter_start(1)
                return 0
            lax.fori_loop(0, PAIRS, _pair, 0)
            scatter_wait(0)
            scatter_wait(1)
            return 0
        lax.fori_loop(0, EP // CHUNK, _chunk, 0)

        pltpu.async_copy(denT, accD.at[idxr], dsem, add=True)
        pltpu.make_async_copy(denT, accD.at[idxr], dsem).wait()

        plsc.subcore_barrier()
        base = (2 * c + j) * NP
        pltpu.sync_copy(accD.at[pl.ds(s * NRT, NRT)], dzv)
        for k in range(NPT // CPY):
            row0 = s * NPT + k * CPY
            pltpu.sync_copy(accS.at[pl.ds(row0, CPY)], zbuf)

            def _div16(g, _):
                dv = dzv[k, pl.ds(g * L, L)]
                rv = 1.0 / (dv + 1e-8)
                for t in range(L):
                    i = g * L + t
                    r0 = rv[t]
                    for q in range(HID // L):
                        zbuf[i, pl.ds(q * L, L)] = (
                            zbuf[i, pl.ds(q * L, L)] * r0)
                return 0
            lax.fori_loop(0, CPY // L, _div16, 0)
            pltpu.sync_copy(zbuf, out_hbm.at[pl.ds(base + row0, CPY)])
        return 0
    lax.fori_loop(0, 2, _pass, 0)


@functools.partial(
    pl.kernel,
    out_type=jax.ShapeDtypeStruct((4 * NP, HID), jnp.float32),
    mesh=_mesh,
    scratch_types=[
        pltpu.VMEM((2 * N,), jnp.float32),
        pltpu.VMEM((CHUNK,), jnp.int32),
        pltpu.VMEM((CHUNK,), jnp.int32),
        pltpu.VMEM((CHUNK,), jnp.float32),
        pltpu.VMEM((B,), jnp.int32),
        pltpu.VMEM((B,), jnp.int32),
        pltpu.VMEM((B,), jnp.float32),
        pltpu.VMEM((B, HID), jnp.float32),
        pltpu.VMEM((B, HID), jnp.float32),
        pltpu.VMEM((B,), jnp.int32),
        pltpu.VMEM((B,), jnp.int32),
        pltpu.VMEM((B,), jnp.float32),
        pltpu.VMEM((B, HID), jnp.float32),
        pltpu.VMEM((B, HID), jnp.float32),
        pltpu.VMEM((CPY, HID), jnp.float32),
        pltpu.VMEM((NRT, DW), jnp.float32),
        pltpu.VMEM((NR, DW), jnp.float32),
        pltpu.VMEM((NR,), jnp.int32),
        pltpu.VMEM_SHARED((NP, HID), jnp.float32),
        pltpu.VMEM_SHARED((NR, DW), jnp.float32),
        pltpu.SemaphoreType.DMA,
        pltpu.SemaphoreType.DMA,
        pltpu.SemaphoreType.DMA,
        pltpu.SemaphoreType.DMA,
        pltpu.SemaphoreType.DMA,
    ],
    compiler_params=_sc_params,
)
def _edge_kernel(*refs):
    _edge_body(*refs)


QPT = Q // (NC * NS)


def _qgather_body(h_hbm, qa_hbm, qb_hbm, oa_hbm, ob_hbm, idxv, buf, sem):
    c = lax.axis_index("c")
    s = lax.axis_index("s")
    base = (s * NC + c) * QPT
    pltpu.sync_copy(qa_hbm.at[pl.ds(base, QPT)], idxv)
    pltpu.async_copy(h_hbm.at[idxv], buf, sem).wait()
    pltpu.sync_copy(buf, oa_hbm.at[pl.ds(base, QPT)])
    pltpu.sync_copy(qb_hbm.at[pl.ds(base, QPT)], idxv)
    pltpu.async_copy(h_hbm.at[idxv], buf, sem).wait()
    pltpu.sync_copy(buf, ob_hbm.at[pl.ds(base, QPT)])


@functools.partial(
    pl.kernel,
    out_type=(jax.ShapeDtypeStruct((Q, HID), jnp.float32),
              jax.ShapeDtypeStruct((Q, HID), jnp.float32)),
    mesh=_mesh,
    scratch_types=[
        pltpu.VMEM((QPT,), jnp.int32),
        pltpu.VMEM((QPT, HID), jnp.float32),
        pltpu.SemaphoreType.DMA,
    ],
    compiler_params=_sc_params,
)
def _qgather_kernel(*refs):
    _qgather_body(*refs)


def _mlp_body(ha_ref, hb_ref, p1wa_ref, p1wb_ref, p1b_ref, p2w_ref, p2b_ref,
              p3w_ref, p3b_ref, out_ref):
    z = jnp.dot(ha_ref[...], p1wa_ref[...], preferred_element_type=jnp.float32)
    z = z + jnp.dot(hb_ref[...], p1wb_ref[...],
                    preferred_element_type=jnp.float32)
    z = jnp.maximum(z + p1b_ref[...], 0.0)
    z = jnp.maximum(jnp.dot(z, p2w_ref[...],
                            preferred_element_type=jnp.float32) + p2b_ref[...],
                    0.0)
    logits = jnp.sum(z * p3w_ref[...], axis=1, keepdims=True)
    out_ref[...] = jax.nn.sigmoid(logits + p3b_ref[0:1, 0:1])


NB = 1000
GRID = N // NB


def _pad8(m):
    return jnp.zeros((8, m.shape[-1]), jnp.float32).at[:m.shape[0]].set(m)


def _enc_body(x_ref, nt_ref, encW_ref, encb_ref, lng_ref, lnb_ref,
              resW_ref, resb_ref, h_ref, hres_ref):
    x = x_ref[...]
    nt = nt_ref[...]
    acc = jnp.zeros((NB, D), jnp.float32)
    for t in range(3):
        y = jnp.dot(x, encW_ref[t], preferred_element_type=jnp.float32)
        y = y + encb_ref[t:t + 1, :]
        mu = jnp.mean(y, axis=-1, keepdims=True)
        dev = y - mu
        var = jnp.mean(dev * dev, axis=-1, keepdims=True)
        yn = dev * jax.lax.rsqrt(var + 1e-5)
        yn = yn * lng_ref[t:t + 1, :] + lnb_ref[t:t + 1, :]
        yn = jnp.maximum(yn, 0.0)
        acc = jnp.where(nt == t, yn, acc)
    h_ref[...] = acc
    hres_ref[...] = (jnp.dot(acc, resW_ref[...],
                             preferred_element_type=jnp.float32)
                     + resb_ref[0:1, :])


def _encode(x, nt, enc_W, enc_b, ln_g, ln_b, res_W, res_b):
    return pl.pallas_call(
        _enc_body,
        grid=(GRID,),
        in_specs=[
            pl.BlockSpec((NB, D), lambda i: (i, 0)),
            pl.BlockSpec((NB, 1), lambda i: (i, 0)),
            pl.BlockSpec((3, D, D), lambda i: (0, 0, 0)),
            pl.BlockSpec((8, D), lambda i: (0, 0)),
            pl.BlockSpec((8, D), lambda i: (0, 0)),
            pl.BlockSpec((8, D), lambda i: (0, 0)),
            pl.BlockSpec((D, HID), lambda i: (0, 0)),
            pl.BlockSpec((8, HID), lambda i: (0, 0)),
        ],
        out_specs=[pl.BlockSpec((NB, D), lambda i: (i, 0)),
                   pl.BlockSpec((NB, HID), lambda i: (i, 0))],
        out_shape=[jax.ShapeDtypeStruct((N, D), jnp.float32),
                   jax.ShapeDtypeStruct((N, HID), jnp.float32)],
    )(x, nt, enc_W, _pad8(enc_b), _pad8(ln_g), _pad8(ln_b), res_W,
      _pad8(res_b[None, :]))


def _pre_body(h_ref, W_ref, a_ref, hp_ref, s_ref):
    h = h_ref[...]
    for hd in range(HEADS):
        y = jnp.dot(h, W_ref[hd], preferred_element_type=jnp.float32)
        hp_ref[hd] = y
        asrc = a_ref[hd:hd + 1, :HID]
        adst = a_ref[hd:hd + 1, HID:]
        s_ref[0, 2 * hd:2 * hd + 1, :] = lax.dot_general(
            asrc, y, (((1,), (1,)), ((), ())))
        s_ref[0, 2 * hd + 1:2 * hd + 2, :] = lax.dot_general(
            adst, y, (((1,), (1,)), ((), ())))


def _layer_pre(h, W, a):
    din = h.shape[1]
    hp4, s8 = pl.pallas_call(
        _pre_body,
        grid=(GRID,),
        in_specs=[
            pl.BlockSpec((NB, din), lambda i: (i, 0)),
            pl.BlockSpec((HEADS, din, HID), lambda i: (0, 0, 0)),
            pl.BlockSpec((8, 2 * HID), lambda i: (0, 0)),
        ],
        out_specs=[pl.BlockSpec((HEADS, NB, HID), lambda i: (0, i, 0)),
                   pl.BlockSpec((1, 8, NB), lambda i: (i, 0, 0))],
        out_shape=[jax.ShapeDtypeStruct((HEADS, N, HID), jnp.float32),
                   jax.ShapeDtypeStruct((GRID, 8, NB), jnp.float32)],
    )(h, W, _pad8(a))
    s8 = s8.transpose(1, 0, 2).reshape(8, N)
    return hp4.reshape(HEADS * N, HID), s8.reshape(2, 4 * N)


def _post_cat_body(acc_ref, h_ref):
    for hd in range(HEADS):
        hn = acc_ref[hd]
        hn = jnp.where(hn > 0, hn, jnp.exp(hn) - 1.0)
        h_ref[:, hd * HID:(hd + 1) * HID] = hn


def _post_mean_body(acc_ref, hres_ref, h_ref):
    total = jnp.zeros((NB, HID), jnp.float32)
    for hd in range(HEADS):
        hn = acc_ref[hd]
        hn = jnp.where(hn > 0, hn, jnp.exp(hn) - 1.0)
        total = total + hn
    h_ref[...] = total * (1.0 / HEADS) + hres_ref[...]


def _layer_post(acc, hres):
    if hres is None:
        return pl.pallas_call(
            _post_cat_body,
            grid=(GRID,),
            in_specs=[pl.BlockSpec((HEADS, NB, HID), lambda i: (0, i, 0))],
            out_specs=pl.BlockSpec((NB, HEADS * HID), lambda i: (i, 0)),
            out_shape=jax.ShapeDtypeStruct((N, HEADS * HID), jnp.float32),
        )(acc)
    return pl.pallas_call(
        _post_mean_body,
        grid=(GRID,),
        in_specs=[pl.BlockSpec((HEADS, NB, HID), lambda i: (0, i, 0)),
                  pl.BlockSpec((NB, HID), lambda i: (i, 0))],
        out_specs=pl.BlockSpec((NB, HID), lambda i: (i, 0)),
        out_shape=jax.ShapeDtypeStruct((N, HID), jnp.float32),
    )(acc, hres)


def kernel(node_features, node_types, edge_index, edge_weights, query_pairs,
           enc_W, enc_b, ln_g, ln_b, W1, a1, W2, a2, W3, a3,
           res_W, res_b, p1_W, p1_b, p2_W, p2_b, p3_W, p3_b):
    src = edge_index[0].astype(jnp.int32)
    dst = edge_index[1].astype(jnp.int32)
    ew = edge_weights.astype(jnp.float32)
    nt = node_types.astype(jnp.int32).reshape(N, 1)

    h, h_residual = _encode(node_features, nt, enc_W, enc_b, ln_g, ln_b,
                            res_W, res_b)

    for (W, a, concat) in ((W1, a1, True), (W2, a2, True), (W3, a3, False)):
        hp4n, s_pack = _layer_pre(h, W, a)
        acc = _edge_kernel(hp4n, s_pack, src, dst, ew)
        acc = acc.reshape(HEADS, NP, HID)
        h = _layer_post(acc, None if concat else h_residual)

    qa = query_pairs[:, 0].astype(jnp.int32)
    qb = query_pairs[:, 1].astype(jnp.int32)
    h_ad, h_cat = _qgather_kernel(h, qa, qb)

    p3b_p = jnp.zeros((8, 128), jnp.float32).at[0, 0].set(p3_b[0])
    out = pl.pallas_call(
        _mlp_body,
        out_shape=jax.ShapeDtypeStruct((Q, 1), jnp.float32),
    )(h_ad, h_cat, p1_W[:HID], p1_W[HID:], p1_b[None, :], p2_W, p2_b[None, :],
      p3_W.reshape(1, HID), p3b_p)
    return out[:, 0]


# speedup vs baseline: 50.5535x; 1.0038x over previous
import functools

import jax
import jax.numpy as jnp
from jax import lax
from jax.experimental import pallas as pl
from jax.experimental.pallas import tpu as pltpu
from jax.experimental.pallas import tpu_sc as plsc

N = 10000
E = 320000
D = 128
HID = 64
HEADS = 4
Q = 4096

NC = 2
NS = 16
L = 16

EP = E // NS
CHUNK = 4000
B = 80
BPC = CHUNK // B
PAIRS = BPC // 2
NP = 10240
NPT = NP // NS
CPY = 64
DW = 64
DSH = 6
NR = NP // DW
NRT = NR // NS
_mesh = plsc.VectorSubcoreMesh(core_axis_name="c", subcore_axis_name="s")
_sc_params = pltpu.CompilerParams(needs_layout_passes=False,
                                  use_tc_tiling_on_sc=False)


def _edge_body(hp_hbm, sp_hbm, src_hbm, dst_hbm, ew_hbm, out_hbm,
               sv, src_c, dst_c, ew_c,
               idxs0, idxd0, expv0, gbuf0, obuf0,
               idxs1, idxd1, expv1, gbuf1, obuf1,
               zbuf, dzv, denT, idxr, accS, accD,
               gsem0, gsem1, ssem0, ssem1, dsem):
    c = lax.axis_index("c")
    s = lax.axis_index("s")

    slots = ((idxs0, idxd0, expv0, gbuf0, obuf0, gsem0, ssem0),
             (idxs1, idxd1, expv1, gbuf1, obuf1, gsem1, ssem1))

    lane = lax.broadcasted_iota(jnp.int32, (L,), 0)
    for q in range(NR // L):
        idxr[pl.ds(q * L, L)] = lane + q * L

    def _pass(j, _):
        pltpu.sync_copy(sp_hbm.at[c, pl.ds(j * 2 * N, 2 * N)], sv)
        def _z(i, _):
            for q in range(HID // L):
                zbuf[i, pl.ds(q * L, L)] = jnp.zeros((L,), jnp.float32)
            return 0
        lax.fori_loop(0, CPY, _z, 0)
        def _zdz(i, _):
            for q in range(DW // L):
                dzv[i, pl.ds(q * L, L)] = jnp.zeros((L,), jnp.float32)
            return 0
        lax.fori_loop(0, NRT, _zdz, 0)
        for k in range(NPT // CPY):
            pltpu.sync_copy(zbuf, accS.at[pl.ds(s * NPT + k * CPY, CPY)])
        pltpu.sync_copy(dzv, accD.at[pl.ds(s * NRT, NRT)])

        def _zd(i, _):
            for q in range(DW // L):
                denT[i, pl.ds(q * L, L)] = jnp.zeros((L,), jnp.float32)
            return 0
        lax.fori_loop(0, NR, _zd, 0)
        plsc.subcore_barrier()

        hb = (2 * c + j) * N

        def scalar_phase(slot, b):
            idxs_, idxd_, expv_ = slots[slot][0], slots[slot][1], slots[slot][2]
            for g in range(B // L):
                i0 = b * B + g * L
                srcv = src_c[pl.ds(i0, L)]
                dstv = dst_c[pl.ds(i0, L)]
                ewv = ew_c[pl.ds(i0, L)]
                idxs_[pl.ds(g * L, L)] = srcv + hb
                idxd_[pl.ds(g * L, L)] = dstv
                ssrc = plsc.load_gather(sv, [srcv])
                sdst = plsc.load_gather(sv, [dstv + N])
                e = ssrc + sdst
                e = jnp.where(e > 0, e, 0.2 * e)
                ex = jnp.exp(e) * (1.0 + jnp.maximum(ewv, 0.0))
                expv_[pl.ds(g * L, L)] = ex
                plsc.addupdate_scatter(
                    denT,
                    [lax.shift_right_logical(dstv, DSH),
                     jnp.bitwise_and(dstv, DW - 1)], ex)

        def gather_start(slot):
            pltpu.async_copy(hp_hbm.at[slots[slot][0]], slots[slot][3],
                             slots[slot][5])

        def gather_wait(slot):
            pltpu.make_async_copy(hp_hbm.at[slots[slot][0]], slots[slot][3],
                                  slots[slot][5]).wait()

        def scale(slot):
            expv_, gbuf_, obuf_ = slots[slot][2], slots[slot][3], slots[slot][4]

            def _row16(g, _):
                ev = expv_[pl.ds(g * L, L)]
                for t in range(L):
                    i = g * L + t
                    e0 = ev[t]
                    for q in range(HID // L):
                        obuf_[i, pl.ds(q * L, L)] = (
                            gbuf_[i, pl.ds(q * L, L)] * e0)
                return 0
            lax.fori_loop(0, B // L, _row16, 0)

        def scatter_start(slot):
            pltpu.async_copy(slots[slot][4], accS.at[slots[slot][1]],
                             slots[slot][6], add=True)

        def scatter_wait(slot):
            pltpu.make_async_copy(slots[slot][4], accS.at[slots[slot][1]],
                                  slots[slot][6]).wait()

        def _chunk(ch, _):
            off = s * EP + ch * CHUNK
            pltpu.sync_copy(src_hbm.at[pl.ds(off, CHUNK)], src_c)
            pltpu.sync_copy(dst_hbm.at[pl.ds(off, CHUNK)], dst_c)
            pltpu.sync_copy(ew_hbm.at[pl.ds(off, CHUNK)], ew_c)

            scalar_phase(0, 0)
            gather_start(0)

            def _pair(t, _):
                b0 = 2 * t
                scalar_phase(1, b0 + 1)
                gather_start(1)
                gather_wait(0)

                @pl.when(t > 0)
                def _():
                    scatter_wait(0)
                scale(0)
                scatter_start(0)

                @pl.when(t < PAIRS - 1)
                def _():
                    scalar_phase(0, b0 + 2)
                    gather_start(0)

                gather_wait(1)

                @pl.when(t > 0)
                def _():
                    scatter_wait(1)
                scale(1)
                scatter_start(1)
                return 0
            lax.fori_loop(0, PAIRS, _pair, 0)
            scatter_wait(0)
            scatter_wait(1)
            return 0
        lax.fori_loop(0, EP // CHUNK, _chunk, 0)

        pltpu.async_copy(denT, accD.at[idxr], dsem, add=True)
        pltpu.make_async_copy(denT, accD.at[idxr], dsem).wait()

        plsc.subcore_barrier()
        base = (2 * c + j) * NP
        pltpu.sync_copy(accD.at[pl.ds(s * NRT, NRT)], dzv)
        for k in range(NPT // CPY):
            row0 = s * NPT + k * CPY
            pltpu.sync_copy(accS.at[pl.ds(row0, CPY)], zbuf)

            def _div16(g, _):
                dv = dzv[k, pl.ds(g * L, L)] + 1e-8
                rv = 1.0 / dv
                rv = rv * (2.0 - dv * rv)
                for t in range(L):
                    i = g * L + t
                    r0 = rv[t]
                    for q in range(HID // L):
                        zbuf[i, pl.ds(q * L, L)] = (
                            zbuf[i, pl.ds(q * L, L)] * r0)
                return 0
            lax.fori_loop(0, CPY // L, _div16, 0)
            pltpu.sync_copy(zbuf, out_hbm.at[pl.ds(base + row0, CPY)])
        return 0
    lax.fori_loop(0, 2, _pass, 0)


@functools.partial(
    pl.kernel,
    out_type=jax.ShapeDtypeStruct((4 * NP, HID), jnp.float32),
    mesh=_mesh,
    scratch_types=[
        pltpu.VMEM((2 * N,), jnp.float32),
        pltpu.VMEM((CHUNK,), jnp.int32),
        pltpu.VMEM((CHUNK,), jnp.int32),
        pltpu.VMEM((CHUNK,), jnp.float32),
        pltpu.VMEM((B,), jnp.int32),
        pltpu.VMEM((B,), jnp.int32),
        pltpu.VMEM((B,), jnp.float32),
        pltpu.VMEM((B, HID), jnp.float32),
        pltpu.VMEM((B, HID), jnp.float32),
        pltpu.VMEM((B,), jnp.int32),
        pltpu.VMEM((B,), jnp.int32),
        pltpu.VMEM((B,), jnp.float32),
        pltpu.VMEM((B, HID), jnp.float32),
        pltpu.VMEM((B, HID), jnp.float32),
        pltpu.VMEM((CPY, HID), jnp.float32),
        pltpu.VMEM((NRT, DW), jnp.float32),
        pltpu.VMEM((NR, DW), jnp.float32),
        pltpu.VMEM((NR,), jnp.int32),
        pltpu.VMEM_SHARED((NP, HID), jnp.float32),
        pltpu.VMEM_SHARED((NR, DW), jnp.float32),
        pltpu.SemaphoreType.DMA,
        pltpu.SemaphoreType.DMA,
        pltpu.SemaphoreType.DMA,
        pltpu.SemaphoreType.DMA,
        pltpu.SemaphoreType.DMA,
    ],
    compiler_params=_sc_params,
)
def _edge_kernel(*refs):
    _edge_body(*refs)


QPT = Q // (NC * NS)


def _qgather_body(h_hbm, qa_hbm, qb_hbm, oa_hbm, ob_hbm, idxv, buf, sem):
    c = lax.axis_index("c")
    s = lax.axis_index("s")
    base = (s * NC + c) * QPT
    pltpu.sync_copy(qa_hbm.at[pl.ds(base, QPT)], idxv)
    pltpu.async_copy(h_hbm.at[idxv], buf, sem).wait()
    pltpu.sync_copy(buf, oa_hbm.at[pl.ds(base, QPT)])
    pltpu.sync_copy(qb_hbm.at[pl.ds(base, QPT)], idxv)
    pltpu.async_copy(h_hbm.at[idxv], buf, sem).wait()
    pltpu.sync_copy(buf, ob_hbm.at[pl.ds(base, QPT)])


@functools.partial(
    pl.kernel,
    out_type=(jax.ShapeDtypeStruct((Q, HID), jnp.float32),
              jax.ShapeDtypeStruct((Q, HID), jnp.float32)),
    mesh=_mesh,
    scratch_types=[
        pltpu.VMEM((QPT,), jnp.int32),
        pltpu.VMEM((QPT, HID), jnp.float32),
        pltpu.SemaphoreType.DMA,
    ],
    compiler_params=_sc_params,
)
def _qgather_kernel(*refs):
    _qgather_body(*refs)


def _mlp_body(ha_ref, hb_ref, p1wa_ref, p1wb_ref, p1b_ref, p2w_ref, p2b_ref,
              p3w_ref, p3b_ref, out_ref):
    z = jnp.dot(ha_ref[...], p1wa_ref[...], preferred_element_type=jnp.float32)
    z = z + jnp.dot(hb_ref[...], p1wb_ref[...],
                    preferred_element_type=jnp.float32)
    z = jnp.maximum(z + p1b_ref[...], 0.0)
    z = jnp.maximum(jnp.dot(z, p2w_ref[...],
                            preferred_element_type=jnp.float32) + p2b_ref[...],
                    0.0)
    logits = jnp.sum(z * p3w_ref[...], axis=1, keepdims=True)
    out_ref[...] = jax.nn.sigmoid(logits + p3b_ref[0:1, 0:1])


NB = 1000
GRID = N // NB


def _pad8(m):
    return jnp.zeros((8, m.shape[-1]), jnp.float32).at[:m.shape[0]].set(m)


def _enc_body(x_ref, nt_ref, encW_ref, encb_ref, lng_ref, lnb_ref,
              resW_ref, resb_ref, h_ref, hres_ref):
    x = x_ref[...]
    nt = nt_ref[...]
    acc = jnp.zeros((NB, D), jnp.float32)
    for t in range(3):
        y = jnp.dot(x, encW_ref[t], preferred_element_type=jnp.float32)
        y = y + encb_ref[t:t + 1, :]
        mu = jnp.mean(y, axis=-1, keepdims=True)
        dev = y - mu
        var = jnp.mean(dev * dev, axis=-1, keepdims=True)
        yn = dev * jax.lax.rsqrt(var + 1e-5)
        yn = yn * lng_ref[t:t + 1, :] + lnb_ref[t:t + 1, :]
        yn = jnp.maximum(yn, 0.0)
        acc = jnp.where(nt == t, yn, acc)
    h_ref[...] = acc
    hres_ref[...] = (jnp.dot(acc, resW_ref[...],
                             preferred_element_type=jnp.float32)
                     + resb_ref[0:1, :])


def _encode(x, nt, enc_W, enc_b, ln_g, ln_b, res_W, res_b):
    return pl.pallas_call(
        _enc_body,
        grid=(GRID,),
        in_specs=[
            pl.BlockSpec((NB, D), lambda i: (i, 0)),
            pl.BlockSpec((NB, 1), lambda i: (i, 0)),
            pl.BlockSpec((3, D, D), lambda i: (0, 0, 0)),
            pl.BlockSpec((8, D), lambda i: (0, 0)),
            pl.BlockSpec((8, D), lambda i: (0, 0)),
            pl.BlockSpec((8, D), lambda i: (0, 0)),
            pl.BlockSpec((D, HID), lambda i: (0, 0)),
            pl.BlockSpec((8, HID), lambda i: (0, 0)),
        ],
        out_specs=[pl.BlockSpec((NB, D), lambda i: (i, 0)),
                   pl.BlockSpec((NB, HID), lambda i: (i, 0))],
        out_shape=[jax.ShapeDtypeStruct((N, D), jnp.float32),
                   jax.ShapeDtypeStruct((N, HID), jnp.float32)],
    )(x, nt, enc_W, _pad8(enc_b), _pad8(ln_g), _pad8(ln_b), res_W,
      _pad8(res_b[None, :]))


def _pre_body(h_ref, W_ref, a_ref, hp_ref, s_ref):
    h = h_ref[...]
    for hd in range(HEADS):
        y = jnp.dot(h, W_ref[hd], preferred_element_type=jnp.float32)
        hp_ref[hd] = y
        asrc = a_ref[hd:hd + 1, :HID]
        adst = a_ref[hd:hd + 1, HID:]
        s_ref[0, 2 * hd:2 * hd + 1, :] = lax.dot_general(
            asrc, y, (((1,), (1,)), ((), ())))
        s_ref[0, 2 * hd + 1:2 * hd + 2, :] = lax.dot_general(
            adst, y, (((1,), (1,)), ((), ())))


def _layer_pre(h, W, a):
    din = h.shape[1]
    hp4, s8 = pl.pallas_call(
        _pre_body,
        grid=(GRID,),
        in_specs=[
            pl.BlockSpec((NB, din), lambda i: (i, 0)),
            pl.BlockSpec((HEADS, din, HID), lambda i: (0, 0, 0)),
            pl.BlockSpec((8, 2 * HID), lambda i: (0, 0)),
        ],
        out_specs=[pl.BlockSpec((HEADS, NB, HID), lambda i: (0, i, 0)),
                   pl.BlockSpec((1, 8, NB), lambda i: (i, 0, 0))],
        out_shape=[jax.ShapeDtypeStruct((HEADS, N, HID), jnp.float32),
                   jax.ShapeDtypeStruct((GRID, 8, NB), jnp.float32)],
    )(h, W, _pad8(a))
    s8 = s8.transpose(1, 0, 2).reshape(8, N)
    return hp4.reshape(HEADS * N, HID), s8.reshape(2, 4 * N)


def _post_cat_body(acc_ref, h_ref):
    for hd in range(HEADS):
        hn = acc_ref[hd]
        hn = jnp.where(hn > 0, hn, jnp.exp(hn) - 1.0)
        h_ref[:, hd * HID:(hd + 1) * HID] = hn


def _post_mean_body(acc_ref, hres_ref, h_ref):
    total = jnp.zeros((NB, HID), jnp.float32)
    for hd in range(HEADS):
        hn = acc_ref[hd]
        hn = jnp.where(hn > 0, hn, jnp.exp(hn) - 1.0)
        total = total + hn
    h_ref[...] = total * (1.0 / HEADS) + hres_ref[...]


def _layer_post(acc, hres):
    if hres is None:
        return pl.pallas_call(
            _post_cat_body,
            grid=(GRID,),
            in_specs=[pl.BlockSpec((HEADS, NB, HID), lambda i: (0, i, 0))],
            out_specs=pl.BlockSpec((NB, HEADS * HID), lambda i: (i, 0)),
            out_shape=jax.ShapeDtypeStruct((N, HEADS * HID), jnp.float32),
        )(acc)
    return pl.pallas_call(
        _post_mean_body,
        grid=(GRID,),
        in_specs=[pl.BlockSpec((HEADS, NB, HID), lambda i: (0, i, 0)),
                  pl.BlockSpec((NB, HID), lambda i: (i, 0))],
        out_specs=pl.BlockSpec((NB, HID), lambda i: (i, 0)),
        out_shape=jax.ShapeDtypeStruct((N, HID), jnp.float32),
    )(acc, hres)


def kernel(node_features, node_types, edge_index, edge_weights, query_pairs,
           enc_W, enc_b, ln_g, ln_b, W1, a1, W2, a2, W3, a3,
           res_W, res_b, p1_W, p1_b, p2_W, p2_b, p3_W, p3_b):
    src = edge_index[0].astype(jnp.int32)
    dst = edge_index[1].astype(jnp.int32)
    ew = edge_weights.astype(jnp.float32)
    nt = node_types.astype(jnp.int32).reshape(N, 1)

    h, h_residual = _encode(node_features, nt, enc_W, enc_b, ln_g, ln_b,
                            res_W, res_b)

    for (W, a, concat) in ((W1, a1, True), (W2, a2, True), (W3, a3, False)):
        hp4n, s_pack = _layer_pre(h, W, a)
        acc = _edge_kernel(hp4n, s_pack, src, dst, ew)
        acc = acc.reshape(HEADS, NP, HID)
        h = _layer_post(acc, None if concat else h_residual)

    qa = query_pairs[:, 0].astype(jnp.int32)
    qb = query_pairs[:, 1].astype(jnp.int32)
    h_ad, h_cat = _qgather_kernel(h, qa, qb)

    p3b_p = jnp.zeros((8, 128), jnp.float32).at[0, 0].set(p3_b[0])
    out = pl.pallas_call(
        _mlp_body,
        out_shape=jax.ShapeDtypeStruct((Q, 1), jnp.float32),
    )(h_ad, h_cat, p1_W[:HID], p1_W[HID:], p1_b[None, :], p2_W, p2_b[None, :],
      p3_W.reshape(1, HID), p3b_p)
    return out[:, 0]
